# Initial kernel scaffold; baseline (speedup 1.0000x reference)
import jax
import jax.numpy as jnp
from jax.experimental import pallas as pl


def kernel(embeddings, cluster_ids, prototypes):
    raise NotImplementedError("write your pallas kernel here")


# speedup vs baseline: 172.9218x; 172.9218x over previous
import functools
import math

import jax
import jax.numpy as jnp
from jax import lax
from jax.experimental import pallas as pl
from jax.experimental.pallas import tpu as pltpu
from jax.experimental.pallas import tpu_sc as plsc

B = 4096
P = 8192
D = 32
TEMP = 0.5
MOM = 0.9
LN_M = math.log(MOM)

RB = 8
BR = B // RB
CB = 8
PC = P // CB

NC = 2
NS = 16
HALF = P // NC
NPASS = 2
PASS_ROWS = HALF // NPASS
ROWS_PT = PASS_ROWS // NS
IDXW = 128
WCHUNK = 1024


def _weights_body(cidr_ref, cidc_ref, emb_ref, we_ref):
    r = pl.program_id(0)
    cidr = cidr_ref[...]
    row_gid = r * BR + lax.broadcasted_iota(jnp.int32, (BR, 1), 0)

    def step(k, acc):
        cidc = cidc_ref[:, pl.ds(k * 1024, 1024)]
        col_gid = k * 1024 + lax.broadcasted_iota(jnp.int32, (1, 1024), 1)
        hit = (cidr == cidc) & (col_gid > row_gid)
        return acc + jnp.sum(jnp.where(hit, 1.0, 0.0), axis=1, keepdims=True)

    occ_after = lax.fori_loop(0, B // 1024, step, jnp.zeros((BR, 1), jnp.float32))
    w = (1.0 - MOM) * jnp.exp(occ_after * LN_M)
    we_ref[...] = emb_ref[...] * w


def _loss_body(emb_ref, proto_ref, cidcol_ref, cidrow_ref,
               loss_ref, decayed_ref,
               sumexp_ref, pos_ref, cnt_ref, lossacc_ref):
    r = pl.program_id(0)
    c = pl.program_id(1)

    emb = emb_ref[...]
    en = emb * lax.rsqrt(jnp.maximum(jnp.sum(emb * emb, axis=1, keepdims=True), 1e-24))
    pr = proto_ref[...]
    pn = pr * lax.rsqrt(jnp.maximum(jnp.sum(pr * pr, axis=1, keepdims=True), 1e-24))
    s = lax.dot_general(en, pn, (((1,), (1,)), ((), ())),
                        preferred_element_type=jnp.float32) * (1.0 / TEMP)

    cid_col = cidcol_ref[...]
    col_gid = c * PC + lax.broadcasted_iota(jnp.int32, (1, PC), 1)
    is_pos = cid_col == col_gid

    prev_se = jnp.where(c == 0, jnp.zeros((BR, 1), jnp.float32), sumexp_ref[...])
    sumexp_ref[...] = prev_se + jnp.sum(jnp.exp(s), axis=1, keepdims=True)
    prev_pos = jnp.where(c == 0, jnp.zeros((BR, 1), jnp.float32), pos_ref[...])
    pos_ref[...] = prev_pos + jnp.sum(jnp.where(is_pos, s, 0.0), axis=1, keepdims=True)

    cid_row = cidrow_ref[...]
    colv = c * PC + lax.broadcasted_iota(jnp.int32, (PC, 1), 0)
    hits = colv == cid_row
    contrib = jnp.sum(jnp.where(hits, 1.0, 0.0), axis=1, keepdims=True)
    prev_cnt = jnp.where(r == 0, jnp.zeros((PC, 1), jnp.float32),
                         cnt_ref[pl.ds(c * PC, PC), :])
    cnt = prev_cnt + contrib
    cnt_ref[pl.ds(c * PC, PC), :] = cnt

    decayed_ref[...] = pr * jnp.exp(cnt * LN_M)

    @pl.when(c == CB - 1)
    def _finish_row_block():
        pos = pos_ref[...]
        se = sumexp_ref[...]
        row_loss = -pos + jnp.log(se - jnp.exp(pos))
        prev = jnp.where(r == 0, jnp.zeros((1, 1), jnp.float32), lossacc_ref[...])
        lossacc_ref[...] = prev + jnp.sum(row_loss, axis=(0, 1), keepdims=True)

    @pl.when((c == CB - 1) & (r == RB - 1))
    def _emit_loss():
        loss_ref[...] = lossacc_ref[...] * (1.0 / B)


_weights_call = pl.pallas_call(
    _weights_body,
    grid=(RB,),
    in_specs=[
        pl.BlockSpec((BR, 1), lambda r: (r, 0)),
        pl.BlockSpec((1, B), lambda r: (0, 0)),
        pl.BlockSpec((BR, D), lambda r: (r, 0)),
    ],
    out_specs=pl.BlockSpec((BR, D), lambda r: (r, 0)),
    out_shape=jax.ShapeDtypeStruct((B, D), jnp.float32),
)

_loss_call = pl.pallas_call(
    _loss_body,
    grid=(RB, CB),
    in_specs=[
        pl.BlockSpec((BR, D), lambda r, c: (r, 0)),
        pl.BlockSpec((PC, D), lambda r, c: (c, 0)),
        pl.BlockSpec((BR, 1), lambda r, c: (r, 0)),
        pl.BlockSpec((1, BR), lambda r, c: (0, r)),
    ],
    out_specs=[
        pl.BlockSpec((1, 1), lambda r, c: (0, 0)),
        pl.BlockSpec((PC, D), lambda r, c: (c, 0)),
    ],
    out_shape=[
        jax.ShapeDtypeStruct((1, 1), jnp.float32),
        jax.ShapeDtypeStruct((P, D), jnp.float32),
    ],
    scratch_shapes=[
        pltpu.VMEM((BR, 1), jnp.float32),
        pltpu.VMEM((BR, 1), jnp.float32),
        pltpu.VMEM((P, 1), jnp.float32),
        pltpu.VMEM((1, 1), jnp.float32),
    ],
)


def _scatter_body(cidrow_ref, we_ref, decayed_ref, out_ref):
    c = pl.program_id(0)
    colv = c * PC + lax.broadcasted_iota(jnp.int32, (PC, 1), 0)
    acc = decayed_ref[...]
    for ch in range(B // 1024):
        cid_row = cidrow_ref[:, pl.ds(ch * 1024, 1024)]
        onehot = jnp.where(colv == cid_row, 1.0, 0.0)
        wec = we_ref[pl.ds(ch * 1024, 1024), :]
        acc = acc + lax.dot_general(onehot, wec, (((1,), (0,)), ((), ())),
                                    preferred_element_type=jnp.float32)
    out_ref[...] = acc


_scatter_call = pl.pallas_call(
    _scatter_body,
    grid=(CB,),
    in_specs=[
        pl.BlockSpec((1, B), lambda c: (0, 0)),
        pl.BlockSpec((B, D), lambda c: (0, 0)),
        pl.BlockSpec((PC, D), lambda c: (c, 0)),
    ],
    out_specs=pl.BlockSpec((PC, D), lambda c: (c, 0)),
    out_shape=jax.ShapeDtypeStruct((P, D), jnp.float32),
)


@functools.cache
def _make_sc_update():
    return pl.kernel(
        _sc_update_body,
        out_type=jax.ShapeDtypeStruct((P, D), jnp.float32),
        mesh=plsc.VectorSubcoreMesh(core_axis_name="c", subcore_axis_name="s",
                                    num_cores=NC, num_subcores=NS),
        scratch_types=[
            pltpu.VMEM((B,), jnp.int32),
            pltpu.VMEM((B // IDXW, IDXW), jnp.int32),
            pltpu.VMEM((WCHUNK, D), jnp.float32),
        ],
    )


def _sc_update_body(decayed_hbm, cid_hbm, we_hbm, out_hbm,
                    idx_v, idxloc_v, we_v):
    pl.run_scoped(
        functools.partial(_sc_update_inner, decayed_hbm, cid_hbm, we_hbm,
                          out_hbm, idx_v, idxloc_v, we_v),
        pltpu.VMEM_SHARED((PASS_ROWS + NS, D), jnp.float32),
    )


def _sc_update_inner(decayed_hbm, cid_hbm, we_hbm, out_hbm,
                     idx_v, idxloc_v, we_v, accum_sh):
    cidx = lax.axis_index("c")
    sidx = lax.axis_index("s")
    region0 = sidx * ROWS_PT
    junk = jnp.full((16,), PASS_ROWS + sidx, jnp.int32)

    pltpu.sync_copy(cid_hbm, idx_v)

    for p in range(NPASS):
        base = cidx * HALF + p * PASS_ROWS
        myrows = base + sidx * ROWS_PT

        pltpu.sync_copy(decayed_hbm.at[pl.ds(myrows, ROWS_PT)],
                        accum_sh.at[pl.ds(region0, ROWS_PT)])

        lo = region0

        def _localize(g, carry):
            cid16 = idx_v[pl.ds(g * 16, 16)]
            loc = cid16 - base
            ok = (loc >= lo) & (loc < lo + ROWS_PT)
            idxloc_v[g // 8, pl.ds((g % 8) * 16, 16)] = jnp.where(ok, loc, junk)
            return carry

        lax.fori_loop(0, B // 16, _localize, 0, unroll=8)

        for ch in range(B // WCHUNK):
            pltpu.sync_copy(we_hbm.at[pl.ds(ch * WCHUNK, WCHUNK)], we_v)
            for k in range(WCHUNK // IDXW):
                r = ch * (WCHUNK // IDXW) + k
                pltpu.sync_copy(we_v.at[pl.ds(k * IDXW, IDXW)],
                                accum_sh.at[idxloc_v.at[r]], add=True)

        pltpu.sync_copy(accum_sh.at[pl.ds(region0, ROWS_PT)],
                        out_hbm.at[pl.ds(myrows, ROWS_PT)])


def kernel(embeddings, cluster_ids, prototypes):
    cid_col = cluster_ids.reshape(B, 1)
    cid_row = cluster_ids.reshape(1, B)
    we = _weights_call(cid_col, cid_row, embeddings)
    loss2d, decayed = _loss_call(embeddings, prototypes, cid_col, cid_row)
    new_protos = _scatter_call(cid_row, we, decayed)
    return loss2d[0, 0], new_protos
